# Initial kernel scaffold; baseline (speedup 1.0000x reference)
import jax
import jax.numpy as jnp
from jax.experimental import pallas as pl


def kernel(x, edge_index, W1l, b1l, W1r, W2l, b2l, W2r, Wfc, bfc):
    raise NotImplementedError("write your pallas kernel here")


# speedup vs baseline: 3.1030x; 3.1030x over previous
import functools
import jax
import jax.numpy as jnp
from jax import lax
from jax.experimental import pallas as pl
from jax.experimental.pallas import tpu as pltpu
from jax.experimental.pallas import tpu_sc as plsc

N = 10000
E = 320000
D = 128
H = 128

NC = 2
NS = 16
NW = NC * NS

CHUNK = 128
CHUNKS_PER_W = 80
EP = NW * CHUNKS_PER_W * CHUNK
NPAD = 10112
ROWS_PER_TILE = NPAD // NS
GROUP = 16


def _sc_aggregate(p, src2d, dst2d, zeros_init, width):
  mesh = plsc.VectorSubcoreMesh(core_axis_name="c", subcore_axis_name="s")

  @functools.partial(
      pl.kernel,
      mesh=mesh,
      out_type=jax.ShapeDtypeStruct((NC, NPAD, width), jnp.float32),
      scratch_types=[
          pltpu.VMEM((GROUP, CHUNK), jnp.int32),
          pltpu.VMEM((GROUP, CHUNK), jnp.int32),
          pltpu.VMEM((CHUNK, width), jnp.float32),
          pltpu.VMEM((CHUNK,), jnp.int32),
          pltpu.VMEM_SHARED((NPAD, width), jnp.float32),
          pltpu.SemaphoreType.DMA,
      ])
  def agg_kernel(p_hbm, src_hbm, dst_hbm, z_hbm, feat_out,
                 src_v, dst_v, rows_v, dst_cur, acc_sp, sem):
    cid = lax.axis_index("c")
    sid = lax.axis_index("s")
    wid = cid * NS + sid
    row0 = sid * ROWS_PER_TILE

    for blk in range(5):
      nrows = 128 if blk < 4 else ROWS_PER_TILE - 512
      r0 = row0 + blk * 128
      pltpu.sync_copy(z_hbm.at[pl.ds(r0, nrows)], rows_v.at[pl.ds(0, nrows)])
      pltpu.sync_copy(rows_v.at[pl.ds(0, nrows)], acc_sp.at[pl.ds(r0, nrows)])

    plsc.subcore_barrier()

    def group_body(g, carry):
      base = wid * CHUNKS_PER_W + g * GROUP
      pltpu.sync_copy(src_hbm.at[pl.ds(base, GROUP)], src_v)
      pltpu.sync_copy(dst_hbm.at[pl.ds(base, GROUP)], dst_v)

      def body(c, carry2):
        gcp = pltpu.async_copy(p_hbm.at[src_v.at[c]], rows_v, sem)
        for j in range(CHUNK // 16):
          dst_cur[pl.ds(j * 16, 16)] = dst_v[c, pl.ds(j * 16, 16)]
        gcp.wait()
        pltpu.sync_copy(rows_v, acc_sp.at[dst_cur], add=True)
        return carry2

      lax.fori_loop(0, GROUP, body, 0)
      return carry

    lax.fori_loop(0, CHUNKS_PER_W // GROUP, group_body, 0)

    plsc.subcore_barrier()

    for blk in range(5):
      nrows = 128 if blk < 4 else ROWS_PER_TILE - 512
      r0 = row0 + blk * 128
      pltpu.sync_copy(acc_sp.at[pl.ds(r0, nrows)], rows_v.at[pl.ds(0, nrows)])
      pltpu.sync_copy(rows_v.at[pl.ds(0, nrows)],
                      feat_out.at[cid, pl.ds(r0, nrows)])

  return agg_kernel(p, src2d, dst2d, zeros_init)


def _sc_degree(dst2d, zeros_init, ones_rows):
  mesh = plsc.VectorSubcoreMesh(core_axis_name="c", subcore_axis_name="s")

  @functools.partial(
      pl.kernel,
      mesh=mesh,
      out_type=jax.ShapeDtypeStruct((NC, NPAD, 128), jnp.float32),
      scratch_types=[
          pltpu.VMEM((GROUP, CHUNK), jnp.int32),
          pltpu.VMEM((CHUNK, 128), jnp.float32),
          pltpu.VMEM((CHUNK,), jnp.int32),
          pltpu.VMEM_SHARED((NPAD, 128), jnp.float32),
      ])
  def deg_kernel(dst_hbm, z_hbm, ones_hbm, deg_out, dst_v, rows_v, dst_cur,
                 deg_sp):
    cid = lax.axis_index("c")
    sid = lax.axis_index("s")
    wid = cid * NS + sid
    row0 = sid * ROWS_PER_TILE

    for blk in range(5):
      nrows = 128 if blk < 4 else ROWS_PER_TILE - 512
      r0 = row0 + blk * 128
      pltpu.sync_copy(z_hbm.at[pl.ds(r0, nrows)], rows_v.at[pl.ds(0, nrows)])
      pltpu.sync_copy(rows_v.at[pl.ds(0, nrows)], deg_sp.at[pl.ds(r0, nrows)])
    pltpu.sync_copy(ones_hbm, rows_v)

    plsc.subcore_barrier()

    def group_body(g, carry):
      base = wid * CHUNKS_PER_W + g * GROUP
      pltpu.sync_copy(dst_hbm.at[pl.ds(base, GROUP)], dst_v)

      def body(c, carry2):
        for j in range(CHUNK // 16):
          dst_cur[pl.ds(j * 16, 16)] = dst_v[c, pl.ds(j * 16, 16)]
        pltpu.sync_copy(rows_v, deg_sp.at[dst_cur], add=True)
        return carry2

      lax.fori_loop(0, GROUP, body, 0)
      return carry

    lax.fori_loop(0, CHUNKS_PER_W // GROUP, group_body, 0)

    plsc.subcore_barrier()

    for blk in range(5):
      nrows = 128 if blk < 4 else ROWS_PER_TILE - 512
      r0 = row0 + blk * 128
      pltpu.sync_copy(deg_sp.at[pl.ds(r0, nrows)], rows_v.at[pl.ds(0, nrows)])
      pltpu.sync_copy(rows_v.at[pl.ds(0, nrows)],
                      deg_out.at[cid, pl.ds(r0, nrows)])

  return deg_kernel(dst2d, zeros_init, ones_rows)


BN = 2000


def _tc_layer1_body(x_r, a0_r, a1_r, d0_r, d1_r, w1l_r, w1r_r, b1l_r, h1_r):
  deg = jnp.maximum(d0_r[...] + d1_r[...], 1.0)
  agg = (a0_r[...] + a1_r[...]) / deg
  m = jnp.dot(agg, w1l_r[...], preferred_element_type=jnp.float32)
  m += jnp.dot(x_r[...], w1r_r[...], preferred_element_type=jnp.float32)
  h1_r[...] = jnp.maximum(m + b1l_r[...], 0.0)


def _tc_layer2_body(h1_r, a0_r, a1_r, d0_r, d1_r, w2l_r, w2r_r, b2l_r,
                    wfc_r, bfc_r, out_r):
  deg = jnp.maximum(d0_r[...] + d1_r[...], 1.0)
  agg = (a0_r[...] + a1_r[...]) / deg
  m = jnp.dot(agg, w2l_r[...], preferred_element_type=jnp.float32)
  m += jnp.dot(h1_r[...], w2r_r[...], preferred_element_type=jnp.float32)
  h2 = jnp.maximum(m + b2l_r[...], 0.0)
  out_r[...] = jnp.sum(h2 * wfc_r[...], axis=1, keepdims=True) + bfc_r[...]


def _row_spec(bn, cols):
  return pl.BlockSpec((bn, cols), lambda i: (i, 0))


def _full_spec(r, c):
  return pl.BlockSpec((r, c), lambda i: (0, 0))


def kernel(x, edge_index, W1l, b1l, W1r, W2l, b2l, W2r, Wfc, bfc):
  src = edge_index[0]
  dst = edge_index[1]
  pad = EP - E
  src2d = jnp.concatenate([src, jnp.zeros((pad,), jnp.int32)]).reshape(-1, CHUNK)
  dst2d = jnp.concatenate([dst, jnp.full((pad,), N, jnp.int32)]).reshape(-1, CHUNK)

  zeros_init = jnp.zeros((NPAD, 128), jnp.float32)

  degp = _sc_degree(dst2d, zeros_init,
                    jnp.ones((CHUNK, 128), jnp.float32))
  d0 = degp[0, :N, 0:1]
  d1 = degp[1, :N, 0:1]

  feat1 = _sc_aggregate(x, src2d, dst2d, zeros_init, D)
  a0 = feat1[0, :N]
  a1 = feat1[1, :N]

  grid = (N // BN,)
  h1 = pl.pallas_call(
      _tc_layer1_body,
      grid=grid,
      in_specs=[
          _row_spec(BN, 128),
          _row_spec(BN, 128),
          _row_spec(BN, 128),
          _row_spec(BN, 1),
          _row_spec(BN, 1),
          _full_spec(128, 128),
          _full_spec(128, 128),
          _full_spec(1, 128),
      ],
      out_specs=_row_spec(BN, 128),
      out_shape=jax.ShapeDtypeStruct((N, 128), jnp.float32),
  )(x, a0, a1, d0, d1, W1l, W1r, b1l.reshape(1, 128))

  feat2 = _sc_aggregate(h1, src2d, dst2d, zeros_init, H)
  q0 = feat2[0, :N]
  q1 = feat2[1, :N]

  out = pl.pallas_call(
      _tc_layer2_body,
      grid=grid,
      in_specs=[
          _row_spec(BN, 128),
          _row_spec(BN, 128),
          _row_spec(BN, 128),
          _row_spec(BN, 1),
          _row_spec(BN, 1),
          _full_spec(128, 128),
          _full_spec(128, 128),
          _full_spec(1, 128),
          _full_spec(1, 128),
          _full_spec(1, 1),
      ],
      out_specs=_row_spec(BN, 1),
      out_shape=jax.ShapeDtypeStruct((N, 1), jnp.float32),
  )(h1, q0, q1, d0, d1, W2l, W2r, b2l.reshape(1, 128),
    Wfc.reshape(1, 128), bfc.reshape(1, 1))

  return out.reshape(N)


# speedup vs baseline: 3.3773x; 1.0884x over previous
import functools
import jax
import jax.numpy as jnp
from jax import lax
from jax.experimental import pallas as pl
from jax.experimental.pallas import tpu as pltpu
from jax.experimental.pallas import tpu_sc as plsc

N = 10000
E = 320000
D = 128
H = 128

NC = 2
NS = 16
NW = NC * NS

CHUNK = 128
CHUNKS_PER_W = 80
EP = NW * CHUNKS_PER_W * CHUNK
NPAD = 10112
ROWS_PER_TILE = NPAD // NS
GROUP = 16


def _sc_aggregate(p, src2d, dst2d, zeros_init, width):
  mesh = plsc.VectorSubcoreMesh(core_axis_name="c", subcore_axis_name="s")

  @functools.partial(
      pl.kernel,
      mesh=mesh,
      out_type=jax.ShapeDtypeStruct((NC, NPAD, width), jnp.float32),
      scratch_types=[
          pltpu.VMEM((GROUP, CHUNK), jnp.int32),
          pltpu.VMEM((GROUP, CHUNK), jnp.int32),
          pltpu.VMEM((CHUNK, width), jnp.float32),
          pltpu.VMEM((CHUNK, width), jnp.float32),
          pltpu.VMEM((CHUNK,), jnp.int32),
          pltpu.VMEM_SHARED((NPAD, width), jnp.float32),
          pltpu.SemaphoreType.DMA,
          pltpu.SemaphoreType.DMA,
      ])
  def agg_kernel(p_hbm, src_hbm, dst_hbm, z_hbm, feat_out,
                 src_v, dst_v, rows_a, rows_b, dst_cur, acc_sp, sem_a, sem_b):
    rows_v = rows_a
    cid = lax.axis_index("c")
    sid = lax.axis_index("s")
    wid = cid * NS + sid
    row0 = sid * ROWS_PER_TILE

    for blk in range(5):
      nrows = 128 if blk < 4 else ROWS_PER_TILE - 512
      r0 = row0 + blk * 128
      pltpu.sync_copy(z_hbm.at[pl.ds(r0, nrows)], rows_v.at[pl.ds(0, nrows)])
      pltpu.sync_copy(rows_v.at[pl.ds(0, nrows)], acc_sp.at[pl.ds(r0, nrows)])

    plsc.subcore_barrier()

    bufs = (rows_a, rows_b)
    sems = (sem_a, sem_b)

    def group_body(g, carry):
      base = wid * CHUNKS_PER_W + g * GROUP
      pltpu.sync_copy(src_hbm.at[pl.ds(base, GROUP)], src_v)
      pltpu.sync_copy(dst_hbm.at[pl.ds(base, GROUP)], dst_v)

      copies = [None, None]
      copies[0] = pltpu.async_copy(p_hbm.at[src_v.at[0]], bufs[0], sems[0])
      for c in range(GROUP):
        b = c % 2
        if c + 1 < GROUP:
          nb = (c + 1) % 2
          copies[nb] = pltpu.async_copy(p_hbm.at[src_v.at[c + 1]],
                                        bufs[nb], sems[nb])
        for j in range(CHUNK // 16):
          dst_cur[pl.ds(j * 16, 16)] = dst_v[c, pl.ds(j * 16, 16)]
        copies[b].wait()
        pltpu.sync_copy(bufs[b], acc_sp.at[dst_cur], add=True)
      return carry

    lax.fori_loop(0, CHUNKS_PER_W // GROUP, group_body, 0)

    plsc.subcore_barrier()

    for blk in range(5):
      nrows = 128 if blk < 4 else ROWS_PER_TILE - 512
      r0 = row0 + blk * 128
      pltpu.sync_copy(acc_sp.at[pl.ds(r0, nrows)], rows_v.at[pl.ds(0, nrows)])
      pltpu.sync_copy(rows_v.at[pl.ds(0, nrows)],
                      feat_out.at[cid, pl.ds(r0, nrows)])

  return agg_kernel(p, src2d, dst2d, zeros_init)


def _sc_degree(dst2d, zeros_init, ones_rows):
  mesh = plsc.VectorSubcoreMesh(core_axis_name="c", subcore_axis_name="s")

  @functools.partial(
      pl.kernel,
      mesh=mesh,
      out_type=jax.ShapeDtypeStruct((NC, NPAD, 128), jnp.float32),
      scratch_types=[
          pltpu.VMEM((GROUP, CHUNK), jnp.int32),
          pltpu.VMEM((CHUNK, 128), jnp.float32),
          pltpu.VMEM((CHUNK,), jnp.int32),
          pltpu.VMEM_SHARED((NPAD, 128), jnp.float32),
      ])
  def deg_kernel(dst_hbm, z_hbm, ones_hbm, deg_out, dst_v, rows_v, dst_cur,
                 deg_sp):
    cid = lax.axis_index("c")
    sid = lax.axis_index("s")
    wid = cid * NS + sid
    row0 = sid * ROWS_PER_TILE

    for blk in range(5):
      nrows = 128 if blk < 4 else ROWS_PER_TILE - 512
      r0 = row0 + blk * 128
      pltpu.sync_copy(z_hbm.at[pl.ds(r0, nrows)], rows_v.at[pl.ds(0, nrows)])
      pltpu.sync_copy(rows_v.at[pl.ds(0, nrows)], deg_sp.at[pl.ds(r0, nrows)])
    pltpu.sync_copy(ones_hbm, rows_v)

    plsc.subcore_barrier()

    def group_body(g, carry):
      base = wid * CHUNKS_PER_W + g * GROUP
      pltpu.sync_copy(dst_hbm.at[pl.ds(base, GROUP)], dst_v)

      def body(c, carry2):
        for j in range(CHUNK // 16):
          dst_cur[pl.ds(j * 16, 16)] = dst_v[c, pl.ds(j * 16, 16)]
        pltpu.sync_copy(rows_v, deg_sp.at[dst_cur], add=True)
        return carry2

      lax.fori_loop(0, GROUP, body, 0)
      return carry

    lax.fori_loop(0, CHUNKS_PER_W // GROUP, group_body, 0)

    plsc.subcore_barrier()

    for blk in range(5):
      nrows = 128 if blk < 4 else ROWS_PER_TILE - 512
      r0 = row0 + blk * 128
      pltpu.sync_copy(deg_sp.at[pl.ds(r0, nrows)], rows_v.at[pl.ds(0, nrows)])
      pltpu.sync_copy(rows_v.at[pl.ds(0, nrows)],
                      deg_out.at[cid, pl.ds(r0, nrows)])

  return deg_kernel(dst2d, zeros_init, ones_rows)


BN = 2000


def _tc_layer1_body(x_r, a0_r, a1_r, d0_r, d1_r, w1l_r, w1r_r, b1l_r, h1_r):
  deg = jnp.maximum(d0_r[...] + d1_r[...], 1.0)
  agg = (a0_r[...] + a1_r[...]) / deg
  m = jnp.dot(agg, w1l_r[...], preferred_element_type=jnp.float32)
  m += jnp.dot(x_r[...], w1r_r[...], preferred_element_type=jnp.float32)
  h1_r[...] = jnp.maximum(m + b1l_r[...], 0.0)


def _tc_layer2_body(h1_r, a0_r, a1_r, d0_r, d1_r, w2l_r, w2r_r, b2l_r,
                    wfc_r, bfc_r, out_r):
  deg = jnp.maximum(d0_r[...] + d1_r[...], 1.0)
  agg = (a0_r[...] + a1_r[...]) / deg
  m = jnp.dot(agg, w2l_r[...], preferred_element_type=jnp.float32)
  m += jnp.dot(h1_r[...], w2r_r[...], preferred_element_type=jnp.float32)
  h2 = jnp.maximum(m + b2l_r[...], 0.0)
  out_r[...] = jnp.sum(h2 * wfc_r[...], axis=1, keepdims=True) + bfc_r[...]


def _row_spec(bn, cols):
  return pl.BlockSpec((bn, cols), lambda i: (i, 0))


def _full_spec(r, c):
  return pl.BlockSpec((r, c), lambda i: (0, 0))


def kernel(x, edge_index, W1l, b1l, W1r, W2l, b2l, W2r, Wfc, bfc):
  src = edge_index[0]
  dst = edge_index[1]
  pad = EP - E
  src2d = jnp.concatenate([src, jnp.zeros((pad,), jnp.int32)]).reshape(-1, CHUNK)
  dst2d = jnp.concatenate([dst, jnp.full((pad,), N, jnp.int32)]).reshape(-1, CHUNK)

  zeros_init = jnp.zeros((NPAD, 128), jnp.float32)

  degp = _sc_degree(dst2d, zeros_init,
                    jnp.ones((CHUNK, 128), jnp.float32))
  d0 = degp[0, :N, 0:1]
  d1 = degp[1, :N, 0:1]

  feat1 = _sc_aggregate(x, src2d, dst2d, zeros_init, D)
  a0 = feat1[0, :N]
  a1 = feat1[1, :N]

  grid = (N // BN,)
  h1 = pl.pallas_call(
      _tc_layer1_body,
      grid=grid,
      in_specs=[
          _row_spec(BN, 128),
          _row_spec(BN, 128),
          _row_spec(BN, 128),
          _row_spec(BN, 1),
          _row_spec(BN, 1),
          _full_spec(128, 128),
          _full_spec(128, 128),
          _full_spec(1, 128),
      ],
      out_specs=_row_spec(BN, 128),
      out_shape=jax.ShapeDtypeStruct((N, 128), jnp.float32),
  )(x, a0, a1, d0, d1, W1l, W1r, b1l.reshape(1, 128))

  feat2 = _sc_aggregate(h1, src2d, dst2d, zeros_init, H)
  q0 = feat2[0, :N]
  q1 = feat2[1, :N]

  out = pl.pallas_call(
      _tc_layer2_body,
      grid=grid,
      in_specs=[
          _row_spec(BN, 128),
          _row_spec(BN, 128),
          _row_spec(BN, 128),
          _row_spec(BN, 1),
          _row_spec(BN, 1),
          _full_spec(128, 128),
          _full_spec(128, 128),
          _full_spec(1, 128),
          _full_spec(1, 128),
          _full_spec(1, 1),
      ],
      out_specs=_row_spec(BN, 1),
      out_shape=jax.ShapeDtypeStruct((N, 1), jnp.float32),
  )(h1, q0, q1, d0, d1, W2l, W2r, b2l.reshape(1, 128),
    Wfc.reshape(1, 128), bfc.reshape(1, 1))

  return out.reshape(N)
